# baseline (speedup 1.0000x reference)
import functools

import jax
import jax.numpy as jnp
from jax.experimental import pallas as pl
from jax.experimental.pallas import tpu as pltpu


def _count_body(off_ref, ni_ref, e0_ref, e1_ref, out_ref, *, max_row):
    ni = ni_ref[...]
    m = jnp.logical_and(e0_ref[...] == ni, e1_ref[...] == ni)
    cnt = jnp.sum(m.astype(jnp.int32), axis=1, keepdims=True)
    cnt = cnt + off_ref[0, 0]
    out_ref[...] = jnp.clip(cnt, 0, max_row)


def _bcast_body(n_ref, params_ref, out_ref, *, rows_per_step):
    i = pl.program_id(1)
    base = i * rows_per_step
    for r in range(rows_per_step):
        idx = n_ref[base + r]
        out_ref[pl.ds(r, 1), :] = params_ref[pl.ds(idx, 1), :]


def kernel(params, node_index, batch_ptr, edge_index, batch_shape):
    batch_size = node_index.shape[0]
    if batch_size == 0:
        return jnp.zeros((0, params.shape[1]), dtype=params.dtype)
    n_rows, n_cols = params.shape
    max_edges = edge_index.shape[1]

    off = jnp.reshape(
        jnp.asarray(batch_shape, jnp.int32) - jnp.int32(batch_size), (1, 1)
    )
    ni = node_index.reshape(batch_size, 1)
    e0 = edge_index[:, :, 0]
    e1 = edge_index[:, :, 1]
    n_edges = pl.pallas_call(
        functools.partial(_count_body, max_row=n_rows - 1),
        out_shape=jax.ShapeDtypeStruct((batch_size, 1), jnp.int32),
        in_specs=[
            pl.BlockSpec(memory_space=pltpu.SMEM),
            pl.BlockSpec((batch_size, 1), lambda: (0, 0)),
            pl.BlockSpec((batch_size, max_edges), lambda: (0, 0)),
            pl.BlockSpec((batch_size, max_edges), lambda: (0, 0)),
        ],
        out_specs=pl.BlockSpec((batch_size, 1), lambda: (0, 0)),
    )(off, ni, e0, e1)
    n_edges = n_edges.reshape(batch_size)

    col_block = min(8192, n_cols)
    ncb = pl.cdiv(n_cols, col_block)
    rows_per_step = 8 if batch_size % 8 == 0 else 1
    nbr = batch_size // rows_per_step
    out = pl.pallas_call(
        functools.partial(_bcast_body, rows_per_step=rows_per_step),
        grid=(ncb, nbr),
        out_shape=jax.ShapeDtypeStruct((batch_size, n_cols), params.dtype),
        in_specs=[
            pl.BlockSpec(memory_space=pltpu.SMEM),
            pl.BlockSpec((n_rows, col_block), lambda j, i: (0, j)),
        ],
        out_specs=pl.BlockSpec((rows_per_step, col_block), lambda j, i: (i, j)),
    )(n_edges, params)
    return out


# speedup vs baseline: 1.5937x; 1.5937x over previous
import functools

import jax
import jax.numpy as jnp
from jax.experimental import pallas as pl
from jax.experimental.pallas import tpu as pltpu


def _count_body(off_ref, ni_ref, e0_ref, e1_ref, out_ref, *, max_row):
    ni = ni_ref[...]
    m = jnp.logical_and(e0_ref[...] == ni, e1_ref[...] == ni)
    cnt = jnp.sum(m.astype(jnp.int32), axis=1, keepdims=True)
    cnt = cnt + off_ref[0, 0]
    out_ref[...] = jnp.clip(cnt, 0, max_row)


def _bcast_body(n_ref, params_ref, out_ref, *, rows_per_step):
    i = pl.program_id(1)
    base = i * rows_per_step
    for r in range(rows_per_step):
        idx = n_ref[base + r]
        out_ref[pl.ds(r, 1), :] = params_ref[pl.ds(idx, 1), :]


def kernel(params, node_index, batch_ptr, edge_index, batch_shape):
    batch_size = node_index.shape[0]
    if batch_size == 0:
        return jnp.zeros((0, params.shape[1]), dtype=params.dtype)
    n_rows, n_cols = params.shape
    max_edges = edge_index.shape[1]

    off = jnp.reshape(
        jnp.asarray(batch_shape, jnp.int32) - jnp.int32(batch_size), (1, 1)
    )
    ni = node_index.reshape(batch_size, 1)
    e0 = edge_index[:, :, 0]
    e1 = edge_index[:, :, 1]
    n_edges = pl.pallas_call(
        functools.partial(_count_body, max_row=n_rows - 1),
        out_shape=jax.ShapeDtypeStruct((batch_size, 1), jnp.int32),
        in_specs=[
            pl.BlockSpec(memory_space=pltpu.SMEM),
            pl.BlockSpec((batch_size, 1), lambda: (0, 0)),
            pl.BlockSpec((batch_size, max_edges), lambda: (0, 0)),
            pl.BlockSpec((batch_size, max_edges), lambda: (0, 0)),
        ],
        out_specs=pl.BlockSpec((batch_size, 1), lambda: (0, 0)),
    )(off, ni, e0, e1)
    n_edges = n_edges.reshape(batch_size)

    col_block = min(8192, n_cols)
    ncb = pl.cdiv(n_cols, col_block)
    rows_per_step = 32 if batch_size % 32 == 0 else 1
    nbr = batch_size // rows_per_step
    out = pl.pallas_call(
        functools.partial(_bcast_body, rows_per_step=rows_per_step),
        grid=(ncb, nbr),
        out_shape=jax.ShapeDtypeStruct((batch_size, n_cols), params.dtype),
        in_specs=[
            pl.BlockSpec(memory_space=pltpu.SMEM),
            pl.BlockSpec((n_rows, col_block), lambda j, i: (0, j)),
        ],
        out_specs=pl.BlockSpec((rows_per_step, col_block), lambda j, i: (i, j)),
    )(n_edges, params)
    return out


# speedup vs baseline: 1.8415x; 1.1555x over previous
import functools

import jax
import jax.numpy as jnp
from jax.experimental import pallas as pl
from jax.experimental.pallas import tpu as pltpu


def _count_body(off_ref, ni_ref, e0_ref, e1_ref, out_ref, *, max_row):
    ni = ni_ref[...]
    m = jnp.logical_and(e0_ref[...] == ni, e1_ref[...] == ni)
    cnt = jnp.sum(m.astype(jnp.int32), axis=1, keepdims=True)
    cnt = cnt + off_ref[0, 0]
    out_ref[...] = jnp.clip(cnt, 0, max_row)


def _bcast_body(n_ref, params_ref, out_ref, *, rows_per_step):
    i = pl.program_id(1)
    base = i * rows_per_step
    for r in range(rows_per_step):
        idx = n_ref[base + r]
        out_ref[pl.ds(r, 1), :] = params_ref[pl.ds(idx, 1), :]


def kernel(params, node_index, batch_ptr, edge_index, batch_shape):
    batch_size = node_index.shape[0]
    if batch_size == 0:
        return jnp.zeros((0, params.shape[1]), dtype=params.dtype)
    n_rows, n_cols = params.shape
    max_edges = edge_index.shape[1]

    off = jnp.reshape(
        jnp.asarray(batch_shape, jnp.int32) - jnp.int32(batch_size), (1, 1)
    )
    ni = node_index.reshape(batch_size, 1)
    e0 = edge_index[:, :, 0]
    e1 = edge_index[:, :, 1]
    n_edges = pl.pallas_call(
        functools.partial(_count_body, max_row=n_rows - 1),
        out_shape=jax.ShapeDtypeStruct((batch_size, 1), jnp.int32),
        in_specs=[
            pl.BlockSpec(memory_space=pltpu.SMEM),
            pl.BlockSpec((batch_size, 1), lambda: (0, 0)),
            pl.BlockSpec((batch_size, max_edges), lambda: (0, 0)),
            pl.BlockSpec((batch_size, max_edges), lambda: (0, 0)),
        ],
        out_specs=pl.BlockSpec((batch_size, 1), lambda: (0, 0)),
    )(off, ni, e0, e1)
    n_edges = n_edges.reshape(batch_size)

    col_block = min(8192, n_cols)
    ncb = pl.cdiv(n_cols, col_block)
    rows_per_step = 128 if batch_size % 128 == 0 else 1
    nbr = batch_size // rows_per_step
    out = pl.pallas_call(
        functools.partial(_bcast_body, rows_per_step=rows_per_step),
        grid=(ncb, nbr),
        out_shape=jax.ShapeDtypeStruct((batch_size, n_cols), params.dtype),
        in_specs=[
            pl.BlockSpec(memory_space=pltpu.SMEM),
            pl.BlockSpec((n_rows, col_block), lambda j, i: (0, j)),
        ],
        out_specs=pl.BlockSpec((rows_per_step, col_block), lambda j, i: (i, j)),
    )(n_edges, params)
    return out


# speedup vs baseline: 1.9017x; 1.0327x over previous
import functools

import jax
import jax.numpy as jnp
from jax.experimental import pallas as pl
from jax.experimental.pallas import tpu as pltpu


def _count_body(off_ref, ni_ref, e0_ref, e1_ref, out_ref, *, max_row):
    ni = ni_ref[...]
    m = jnp.logical_and(e0_ref[...] == ni, e1_ref[...] == ni)
    cnt = jnp.sum(m.astype(jnp.int32), axis=1, keepdims=True)
    cnt = cnt + off_ref[0, 0]
    out_ref[...] = jnp.clip(cnt, 0, max_row)


def _bcast_body(n_ref, params_ref, out_ref, *, rows_per_step):
    i = pl.program_id(1)
    base = i * rows_per_step
    for r in range(rows_per_step):
        idx = n_ref[base + r]
        out_ref[pl.ds(r, 1), :] = params_ref[pl.ds(idx, 1), :]


def kernel(params, node_index, batch_ptr, edge_index, batch_shape):
    batch_size = node_index.shape[0]
    if batch_size == 0:
        return jnp.zeros((0, params.shape[1]), dtype=params.dtype)
    n_rows, n_cols = params.shape
    max_edges = edge_index.shape[1]

    off = jnp.reshape(
        jnp.asarray(batch_shape, jnp.int32) - jnp.int32(batch_size), (1, 1)
    )
    ni = node_index.reshape(batch_size, 1)
    e0 = edge_index[:, :, 0]
    e1 = edge_index[:, :, 1]
    n_edges = pl.pallas_call(
        functools.partial(_count_body, max_row=n_rows - 1),
        out_shape=jax.ShapeDtypeStruct((batch_size, 1), jnp.int32),
        in_specs=[
            pl.BlockSpec(memory_space=pltpu.SMEM),
            pl.BlockSpec((batch_size, 1), lambda: (0, 0)),
            pl.BlockSpec((batch_size, max_edges), lambda: (0, 0)),
            pl.BlockSpec((batch_size, max_edges), lambda: (0, 0)),
        ],
        out_specs=pl.BlockSpec((batch_size, 1), lambda: (0, 0)),
    )(off, ni, e0, e1)
    n_edges = n_edges.reshape(batch_size)

    col_block = min(8192, n_cols)
    ncb = pl.cdiv(n_cols, col_block)
    rows_per_step = 256 if batch_size % 256 == 0 else 1
    nbr = batch_size // rows_per_step
    out = pl.pallas_call(
        functools.partial(_bcast_body, rows_per_step=rows_per_step),
        grid=(ncb, nbr),
        out_shape=jax.ShapeDtypeStruct((batch_size, n_cols), params.dtype),
        in_specs=[
            pl.BlockSpec(memory_space=pltpu.SMEM),
            pl.BlockSpec((n_rows, col_block), lambda j, i: (0, j)),
        ],
        out_specs=pl.BlockSpec((rows_per_step, col_block), lambda j, i: (i, j)),
    )(n_edges, params)
    return out


# speedup vs baseline: 3.2866x; 1.7282x over previous
import functools

import jax
import jax.numpy as jnp
from jax import lax
from jax.experimental import pallas as pl
from jax.experimental.pallas import tpu as pltpu


def _count_body(off_ref, ni_ref, e0_ref, e1_ref, out_ref, *, max_row):
    ni = ni_ref[...]
    m = jnp.logical_and(e0_ref[...] == ni, e1_ref[...] == ni)
    cnt = jnp.sum(m.astype(jnp.int32), axis=1, keepdims=True)
    cnt = cnt + off_ref[0, 0]
    out_ref[...] = jnp.clip(cnt, 0, max_row)


def _mm_body(n2_ref, params_ref, outT_ref, hi_ref, lo_ref, *, n_rows, r_blk):
    i = pl.program_id(1)

    @pl.when(i == 0)
    def _():
        p = params_ref[...]
        hi = p.astype(jnp.bfloat16)
        hi_ref[...] = hi
        lo_ref[...] = (p - hi.astype(jnp.float32)).astype(jnp.bfloat16)

    n = n2_ref[...]
    k = lax.broadcasted_iota(jnp.int32, (n_rows, r_blk), 0)
    ohT = (k == n).astype(jnp.bfloat16)
    dn = (((0,), (0,)), ((), ()))
    acc = lax.dot_general(hi_ref[...], ohT, dn, preferred_element_type=jnp.float32)
    acc = acc + lax.dot_general(lo_ref[...], ohT, dn, preferred_element_type=jnp.float32)
    outT_ref[...] = acc


def kernel(params, node_index, batch_ptr, edge_index, batch_shape):
    batch_size = node_index.shape[0]
    if batch_size == 0:
        return jnp.zeros((0, params.shape[1]), dtype=params.dtype)
    n_rows, n_cols = params.shape
    max_edges = edge_index.shape[1]

    off = jnp.reshape(
        jnp.asarray(batch_shape, jnp.int32) - jnp.int32(batch_size), (1, 1)
    )
    ni = node_index.reshape(batch_size, 1)
    e0 = edge_index[:, :, 0]
    e1 = edge_index[:, :, 1]
    n_edges = pl.pallas_call(
        functools.partial(_count_body, max_row=n_rows - 1),
        out_shape=jax.ShapeDtypeStruct((batch_size, 1), jnp.int32),
        in_specs=[
            pl.BlockSpec(memory_space=pltpu.SMEM),
            pl.BlockSpec((batch_size, 1), lambda: (0, 0)),
            pl.BlockSpec((batch_size, max_edges), lambda: (0, 0)),
            pl.BlockSpec((batch_size, max_edges), lambda: (0, 0)),
        ],
        out_specs=pl.BlockSpec((batch_size, 1), lambda: (0, 0)),
    )(off, ni, e0, e1)
    n2 = n_edges.reshape(1, batch_size)

    col_block = min(8192, n_cols)
    ncb = pl.cdiv(n_cols, col_block)
    r_blk = 256 if batch_size % 256 == 0 else batch_size
    nbr = batch_size // r_blk
    out_t = pl.pallas_call(
        functools.partial(_mm_body, n_rows=n_rows, r_blk=r_blk),
        grid=(ncb, nbr),
        out_shape=jax.ShapeDtypeStruct((n_cols, batch_size), params.dtype),
        in_specs=[
            pl.BlockSpec((1, r_blk), lambda j, i: (0, i)),
            pl.BlockSpec((n_rows, col_block), lambda j, i: (0, j)),
        ],
        out_specs=pl.BlockSpec((col_block, r_blk), lambda j, i: (j, i)),
        scratch_shapes=[
            pltpu.VMEM((n_rows, col_block), jnp.bfloat16),
            pltpu.VMEM((n_rows, col_block), jnp.bfloat16),
        ],
    )(n2, params)
    return out_t.T


# speedup vs baseline: 3.8099x; 1.1592x over previous
import functools

import jax
import jax.numpy as jnp
from jax import lax
from jax.experimental import pallas as pl
from jax.experimental.pallas import tpu as pltpu


def _count_body(off_ref, ni_ref, e0_ref, e1_ref, out_ref, *, max_row):
    ni = ni_ref[...]
    m = jnp.logical_and(e0_ref[...] == ni, e1_ref[...] == ni)
    cnt = jnp.sum(m.astype(jnp.int32), axis=1, keepdims=True)
    cnt = cnt + off_ref[0, 0]
    out_ref[...] = jnp.clip(cnt, 0, max_row)


def _mm_body(n2_ref, params_ref, outT_ref, hi_ref, lo_ref, *, n_rows, r_blk):
    i = pl.program_id(1)

    @pl.when(i == 0)
    def _():
        p = params_ref[...]
        hi = p.astype(jnp.bfloat16)
        hi_ref[...] = hi
        lo_ref[...] = (p - hi.astype(jnp.float32)).astype(jnp.bfloat16)

    n = n2_ref[...]
    k = lax.broadcasted_iota(jnp.int32, (n_rows, r_blk), 0)
    ohT = (k == n).astype(jnp.bfloat16)
    dn = (((0,), (0,)), ((), ()))
    acc = lax.dot_general(hi_ref[...], ohT, dn, preferred_element_type=jnp.float32)
    acc = acc + lax.dot_general(lo_ref[...], ohT, dn, preferred_element_type=jnp.float32)
    outT_ref[...] = acc


def kernel(params, node_index, batch_ptr, edge_index, batch_shape):
    batch_size = node_index.shape[0]
    if batch_size == 0:
        return jnp.zeros((0, params.shape[1]), dtype=params.dtype)
    n_rows, n_cols = params.shape
    max_edges = edge_index.shape[1]

    off = jnp.reshape(
        jnp.asarray(batch_shape, jnp.int32) - jnp.int32(batch_size), (1, 1)
    )
    ni = node_index.reshape(batch_size, 1)
    e0 = edge_index[:, :, 0]
    e1 = edge_index[:, :, 1]
    n_edges = pl.pallas_call(
        functools.partial(_count_body, max_row=n_rows - 1),
        out_shape=jax.ShapeDtypeStruct((batch_size, 1), jnp.int32),
        in_specs=[
            pl.BlockSpec(memory_space=pltpu.SMEM),
            pl.BlockSpec((batch_size, 1), lambda: (0, 0)),
            pl.BlockSpec((batch_size, max_edges), lambda: (0, 0)),
            pl.BlockSpec((batch_size, max_edges), lambda: (0, 0)),
        ],
        out_specs=pl.BlockSpec((batch_size, 1), lambda: (0, 0)),
    )(off, ni, e0, e1)
    n2 = n_edges.reshape(1, batch_size)

    col_block = min(4096, n_cols)
    ncb = pl.cdiv(n_cols, col_block)
    r_blk = 512 if batch_size % 512 == 0 else batch_size
    nbr = batch_size // r_blk
    out_t = pl.pallas_call(
        functools.partial(_mm_body, n_rows=n_rows, r_blk=r_blk),
        grid=(ncb, nbr),
        out_shape=jax.ShapeDtypeStruct((n_cols, batch_size), params.dtype),
        in_specs=[
            pl.BlockSpec((1, r_blk), lambda j, i: (0, i)),
            pl.BlockSpec((n_rows, col_block), lambda j, i: (0, j)),
        ],
        out_specs=pl.BlockSpec((col_block, r_blk), lambda j, i: (j, i)),
        scratch_shapes=[
            pltpu.VMEM((n_rows, col_block), jnp.bfloat16),
            pltpu.VMEM((n_rows, col_block), jnp.bfloat16),
        ],
    )(n2, params)
    return out_t.T


# speedup vs baseline: 4.1804x; 1.0972x over previous
import functools

import jax
import jax.numpy as jnp
from jax import lax
from jax.experimental import pallas as pl
from jax.experimental.pallas import tpu as pltpu


def _count_body(off_ref, ni_ref, e0_ref, e1_ref, out_ref, *, max_row):
    ni = ni_ref[...]
    m = jnp.logical_and(e0_ref[...] == ni, e1_ref[...] == ni)
    cnt = jnp.sum(m.astype(jnp.int32), axis=1, keepdims=True)
    cnt = cnt + off_ref[0, 0]
    out_ref[...] = jnp.clip(cnt, 0, max_row)


def _mm_body(n2_ref, params_ref, outT_ref, hi_ref, lo_ref, *, n_rows, r_blk):
    i = pl.program_id(1)

    @pl.when(i == 0)
    def _():
        p = params_ref[...]
        hi = p.astype(jnp.bfloat16)
        hi_ref[...] = hi
        lo_ref[...] = (p - hi.astype(jnp.float32)).astype(jnp.bfloat16)

    n = n2_ref[...]
    k = lax.broadcasted_iota(jnp.int32, (n_rows, r_blk), 0)
    ohT = (k == n).astype(jnp.bfloat16)
    dn = (((0,), (0,)), ((), ()))
    acc = lax.dot_general(hi_ref[...], ohT, dn, preferred_element_type=jnp.float32)
    acc = acc + lax.dot_general(lo_ref[...], ohT, dn, preferred_element_type=jnp.float32)
    outT_ref[...] = acc


def kernel(params, node_index, batch_ptr, edge_index, batch_shape):
    batch_size = node_index.shape[0]
    if batch_size == 0:
        return jnp.zeros((0, params.shape[1]), dtype=params.dtype)
    n_rows, n_cols = params.shape
    max_edges = edge_index.shape[1]

    off = jnp.reshape(
        jnp.asarray(batch_shape, jnp.int32) - jnp.int32(batch_size), (1, 1)
    )
    ni = node_index.reshape(batch_size, 1)
    e0 = edge_index[:, :, 0]
    e1 = edge_index[:, :, 1]
    n_edges = pl.pallas_call(
        functools.partial(_count_body, max_row=n_rows - 1),
        out_shape=jax.ShapeDtypeStruct((batch_size, 1), jnp.int32),
        in_specs=[
            pl.BlockSpec(memory_space=pltpu.SMEM),
            pl.BlockSpec((batch_size, 1), lambda: (0, 0)),
            pl.BlockSpec((batch_size, max_edges), lambda: (0, 0)),
            pl.BlockSpec((batch_size, max_edges), lambda: (0, 0)),
        ],
        out_specs=pl.BlockSpec((batch_size, 1), lambda: (0, 0)),
    )(off, ni, e0, e1)
    n2 = n_edges.reshape(1, batch_size)

    col_block = min(2048, n_cols)
    ncb = pl.cdiv(n_cols, col_block)
    r_blk = 1024 if batch_size % 1024 == 0 else batch_size
    nbr = batch_size // r_blk
    out_t = pl.pallas_call(
        functools.partial(_mm_body, n_rows=n_rows, r_blk=r_blk),
        grid=(ncb, nbr),
        out_shape=jax.ShapeDtypeStruct((n_cols, batch_size), params.dtype),
        in_specs=[
            pl.BlockSpec((1, r_blk), lambda j, i: (0, i)),
            pl.BlockSpec((n_rows, col_block), lambda j, i: (0, j)),
        ],
        out_specs=pl.BlockSpec((col_block, r_blk), lambda j, i: (j, i)),
        scratch_shapes=[
            pltpu.VMEM((n_rows, col_block), jnp.bfloat16),
            pltpu.VMEM((n_rows, col_block), jnp.bfloat16),
        ],
    )(n2, params)
    return out_t.T
